# baseline (speedup 1.0000x reference)
import functools
import math

import jax
import jax.numpy as jnp
from jax import lax
from jax.experimental import pallas as pl
from jax.experimental.pallas import tpu as pltpu
from jax.experimental.pallas import tpu_sc as plsc

_B = 262144
_D = 64
_K = 8
_MIN_SCALE = 1e-10
_LOG2PI = math.log(2.0 * math.pi)
_LN2 = math.log(2.0)

_NW = 32
_B_SC = 49152
_B_TC = _B - _B_SC
_ROWS_PER_W = _B_SC // _NW
_T = 512
_NT = _ROWS_PER_W // _T
_BLK2 = 8192

_LOG_POLY = (
    3.3423269089893903e-08, 1.000003098647089, -0.5000129330593959,
    0.33304812395033884, -0.24911210645380452, 0.2061178523941565,
    -0.18627697325890152, 0.11448435453731831,
)


def _prep_body(locs_ref, logscales_ref, logcoefs_ref, ab_ref, c_ref):
    locs = locs_ref[...]
    logscales = logscales_ref[...]
    logcoefs = logcoefs_ref[...]
    scale = jnp.exp(logscales) + _MIN_SCALE
    inv2 = 1.0 / (scale * scale)
    ab_ref[...] = jnp.concatenate([-0.5 * inv2, locs * inv2], axis=1)
    lc = logcoefs[0] - jax.nn.logsumexp(logcoefs[0])
    c = (lc
         - jnp.sum(jnp.log(scale), axis=1)
         - 0.5 * _D * _LOG2PI
         - 0.5 * jnp.sum(locs * locs * inv2, axis=1))
    c_ref[...] = jnp.concatenate([c, jnp.zeros((8,), jnp.float32)])[None, :]


def _prep(locs, logscales, logcoefs):
    return pl.pallas_call(
        _prep_body,
        out_shape=(
            jax.ShapeDtypeStruct((_K, 2 * _D), jnp.float32),
            jax.ShapeDtypeStruct((1, 16), jnp.float32),
        ),
    )(locs, logscales, logcoefs)


def _poly_log(s):
    bits = lax.bitcast_convert_type(s, jnp.int32)
    e = lax.shift_right_logical(bits, 23) - 127
    mant = lax.bitwise_or(lax.bitwise_and(bits, 0x007FFFFF), 0x3F800000)
    u = lax.bitcast_convert_type(mant, jnp.float32)
    big = u > 1.4142135
    u = jnp.where(big, u * 0.5, u)
    e = jnp.where(big, e + 1, e)
    w = u - 1.0
    p = jnp.full_like(w, _LOG_POLY[-1])
    for coef in _LOG_POLY[-2::-1]:
        p = p * w + coef
    return e.astype(jnp.float32) * _LN2 + p


def _sc_body(x_hbm, ab_hbm, c_hbm, out_hbm,
             xv0, xv1, outv, ab_sp, c_sp, ab_sm, c_sm, sem0, sem1):
    cid = lax.axis_index("c")
    sid = lax.axis_index("s")
    wid = sid * 2 + cid
    base = wid * _ROWS_PER_W

    @pl.when(sid == 0)
    def _():
        pltpu.sync_copy(ab_hbm, ab_sp)
        pltpu.sync_copy(c_hbm, c_sp)

    plsc.subcore_barrier()
    pltpu.sync_copy(ab_sp, ab_sm)
    pltpu.sync_copy(c_sp, c_sm)

    iota = lax.iota(jnp.int32, 16)
    bufs = (xv0, xv1)
    sems = (sem0, sem1)

    def start(t):
        return pltpu.async_copy(
            x_hbm.at[pl.ds((base + t * _T) * _D, _T * _D)],
            bufs[t % 2], sems[t % 2])

    pending = start(0)
    for t in range(_NT):
        pending.wait()
        if t + 1 < _NT:
            pending = start(t + 1)
        buf = bufs[t % 2]

        def group(g, carry):
            ridx = (g * 16 + iota) * _D

            def dstep(d, accs):
                xv = plsc.load_gather(buf, [ridx + d])
                x2 = xv * xv
                return tuple(
                    accs[i] + ab_sm[i, d] * x2 + ab_sm[i, _D + d] * xv
                    for i in range(_K))

            init = tuple(jnp.full((16,), c_sm[0, i], jnp.float32)
                         for i in range(_K))
            accs = lax.fori_loop(0, _D, dstep, init, unroll=8)
            m01 = jnp.maximum(accs[0], accs[1])
            m23 = jnp.maximum(accs[2], accs[3])
            m45 = jnp.maximum(accs[4], accs[5])
            m67 = jnp.maximum(accs[6], accs[7])
            m = jnp.maximum(jnp.maximum(m01, m23), jnp.maximum(m45, m67))
            s = jnp.zeros((16,), jnp.float32)
            for i in range(_K):
                s = s + jnp.exp(accs[i] - m)
            outv[pl.ds(g * 16, 16)] = m + _poly_log(s)
            return carry

        lax.fori_loop(0, _T // 16, group, 0)
        pltpu.sync_copy(outv, out_hbm.at[pl.ds(base + t * _T, _T)])


@functools.partial(
    pl.kernel,
    out_type=jax.ShapeDtypeStruct((_B_SC,), jnp.float32),
    mesh=plsc.VectorSubcoreMesh(core_axis_name="c", subcore_axis_name="s"),
    compiler_params=pltpu.CompilerParams(needs_layout_passes=False),
    scratch_types=[
        pltpu.VMEM((_T * _D,), jnp.float32),
        pltpu.VMEM((_T * _D,), jnp.float32),
        pltpu.VMEM((_T,), jnp.float32),
        pltpu.VMEM_SHARED((_K, 2 * _D), jnp.float32),
        pltpu.VMEM_SHARED((1, 16), jnp.float32),
        pltpu.SMEM((_K, 2 * _D), jnp.float32),
        pltpu.SMEM((1, 16), jnp.float32),
        pltpu.SemaphoreType.DMA,
        pltpu.SemaphoreType.DMA,
    ],
)
def _sc_kernel(x_hbm, ab_hbm, c_hbm, out_hbm, *scratch):
    _sc_body(x_hbm, ab_hbm, c_hbm, out_hbm, *scratch)


def _tc_body(x_ref, locs_ref, logscales_ref, logcoefs_ref, oute_ref, outo_ref):
    locs = locs_ref[...]
    logscales = logscales_ref[...]
    logcoefs = logcoefs_ref[...]

    scale = jnp.exp(logscales) + _MIN_SCALE
    inv2 = 1.0 / (scale * scale)
    a = -0.5 * inv2
    t = locs * inv2
    lc = logcoefs[0] - jax.nn.logsumexp(logcoefs[0])
    c = (lc
         - jnp.sum(jnp.log(scale), axis=1)
         - 0.5 * _D * _LOG2PI
         - 0.5 * jnp.sum(locs * locs * inv2, axis=1))

    xp = x_ref[...]
    g = jnp.concatenate([xp, xp * xp], axis=1)
    z = jnp.zeros((_K, _D), jnp.float32)
    w = jnp.concatenate([
        jnp.concatenate([t, z, a, z], axis=1),
        jnp.concatenate([z, t, z, a], axis=1),
    ], axis=0)
    nt = (((1,), (1,)), ((), ()))
    lp = (lax.dot_general(w, g, nt, preferred_element_type=jnp.float32)
          + jnp.concatenate([c, c])[:, None])
    for lp_half, out_ref in ((lp[:_K], oute_ref), (lp[_K:], outo_ref)):
        m = jnp.max(lp_half, axis=0)
        s = jnp.sum(jnp.exp(lp_half - m[None, :]), axis=0)
        out_ref[...] = (m + jnp.log(s))[None, None, :]


def _tc_part(xp, locs, logscales, logcoefs):
    half = xp.shape[0]
    grid = half // _BLK2
    oute, outo = pl.pallas_call(
        _tc_body,
        grid=(grid,),
        in_specs=[
            pl.BlockSpec((_BLK2, 2 * _D), lambda i: (i, 0)),
            pl.BlockSpec((_K, _D), lambda i: (0, 0)),
            pl.BlockSpec((_K, _D), lambda i: (0, 0)),
            pl.BlockSpec((1, _K), lambda i: (0, 0)),
        ],
        out_specs=(
            pl.BlockSpec((1, 1, _BLK2), lambda i: (i, 0, 0)),
            pl.BlockSpec((1, 1, _BLK2), lambda i: (i, 0, 0)),
        ),
        out_shape=(
            jax.ShapeDtypeStruct((grid, 1, _BLK2), jnp.float32),
            jax.ShapeDtypeStruct((grid, 1, _BLK2), jnp.float32),
        ),
    )(xp, locs, logscales, logcoefs)
    return jnp.stack([oute.reshape(half), outo.reshape(half)],
                     axis=-1).reshape(2 * half)


@jax.jit
def kernel(x, locs, logscales, logcoefs):
    xp = x.reshape(_B // 2, 2 * _D)
    return _tc_part(xp, locs, logscales, logcoefs)


# speedup vs baseline: 1.2728x; 1.2728x over previous
import functools
import math

import jax
import jax.numpy as jnp
from jax import lax
from jax.experimental import pallas as pl
from jax.experimental.pallas import tpu as pltpu
from jax.experimental.pallas import tpu_sc as plsc

_B = 262144
_D = 64
_K = 8
_MIN_SCALE = 1e-10
_LOG2PI = math.log(2.0 * math.pi)
_LN2 = math.log(2.0)

_NW = 32
_B_SC = 49152
_B_TC = _B - _B_SC
_ROWS_PER_W = _B_SC // _NW
_T = 256
_NT = _ROWS_PER_W // _T
_BLK = 16384

_LOG_POLY = (
    3.3423269089893903e-08, 1.000003098647089, -0.5000129330593959,
    0.33304812395033884, -0.24911210645380452, 0.2061178523941565,
    -0.18627697325890152, 0.11448435453731831,
)


def _prep_body(locs_ref, logscales_ref, logcoefs_ref, ab_ref, c_ref):
    locs = locs_ref[...]
    logscales = logscales_ref[...]
    logcoefs = logcoefs_ref[...]
    scale = jnp.exp(logscales) + _MIN_SCALE
    inv2 = 1.0 / (scale * scale)
    ab_ref[...] = jnp.concatenate([-0.5 * inv2, locs * inv2], axis=1)
    lc = logcoefs[0] - jax.nn.logsumexp(logcoefs[0])
    c = (lc
         - jnp.sum(jnp.log(scale), axis=1)
         - 0.5 * _D * _LOG2PI
         - 0.5 * jnp.sum(locs * locs * inv2, axis=1))
    c_ref[...] = jnp.concatenate([c, jnp.zeros((8,), jnp.float32)])[None, :]


def _prep(locs, logscales, logcoefs):
    return pl.pallas_call(
        _prep_body,
        out_shape=(
            jax.ShapeDtypeStruct((_K, 2 * _D), jnp.float32),
            jax.ShapeDtypeStruct((1, 16), jnp.float32),
        ),
    )(locs, logscales, logcoefs)


def _poly_log(s):
    bits = lax.bitcast_convert_type(s, jnp.int32)
    e = lax.shift_right_logical(bits, 23) - 127
    mant = lax.bitwise_or(lax.bitwise_and(bits, 0x007FFFFF), 0x3F800000)
    u = lax.bitcast_convert_type(mant, jnp.float32)
    big = u > 1.4142135
    u = jnp.where(big, u * 0.5, u)
    e = jnp.where(big, e + 1, e)
    w = u - 1.0
    p = jnp.full_like(w, _LOG_POLY[-1])
    for coef in _LOG_POLY[-2::-1]:
        p = p * w + coef
    return e.astype(jnp.float32) * _LN2 + p


def _sc_body(x_hbm, ab_hbm, c_hbm, out_hbm,
             xv0, xv1, outv, ab_sp, c_sp, ab_sm, c_sm, sem0, sem1):
    cid = lax.axis_index("c")
    sid = lax.axis_index("s")
    wid = sid * 2 + cid
    base = _B_TC + wid * _ROWS_PER_W

    @pl.when(sid == 0)
    def _():
        pltpu.sync_copy(ab_hbm, ab_sp)
        pltpu.sync_copy(c_hbm, c_sp)

    plsc.subcore_barrier()
    pltpu.sync_copy(ab_sp, ab_sm)
    pltpu.sync_copy(c_sp, c_sm)

    iota = lax.iota(jnp.int32, 16)
    bufs = (xv0, xv1)
    sems = (sem0, sem1)

    def start(t):
        return pltpu.async_copy(
            x_hbm.at[pl.ds(base + t * _T, _T), :],
            bufs[t % 2], sems[t % 2])

    pending = start(0)
    for t in range(_NT):
        pending.wait()
        if t + 1 < _NT:
            pending = start(t + 1)
        buf = bufs[t % 2]

        def group(g, carry):
            rows = g * 16 + iota

            def dstep(d, accs):
                cols = jnp.full((16,), d, jnp.int32)
                xv = plsc.load_gather(buf, [rows, cols])
                x2 = xv * xv
                return tuple(
                    accs[i] + ab_sm[i, d] * x2 + ab_sm[i, _D + d] * xv
                    for i in range(_K))

            init = tuple(jnp.full((16,), c_sm[0, i], jnp.float32)
                         for i in range(_K))
            accs = lax.fori_loop(0, _D, dstep, init, unroll=8)
            m01 = jnp.maximum(accs[0], accs[1])
            m23 = jnp.maximum(accs[2], accs[3])
            m45 = jnp.maximum(accs[4], accs[5])
            m67 = jnp.maximum(accs[6], accs[7])
            m = jnp.maximum(jnp.maximum(m01, m23), jnp.maximum(m45, m67))
            s = jnp.zeros((16,), jnp.float32)
            for i in range(_K):
                s = s + jnp.exp(accs[i] - m)
            outv[pl.ds(g * 16, 16)] = m + _poly_log(s)
            return carry

        lax.fori_loop(0, _T // 16, group, 0)
        pltpu.sync_copy(outv, out_hbm.at[pl.ds(wid * _ROWS_PER_W + t * _T, _T)])


@functools.partial(
    pl.kernel,
    out_type=jax.ShapeDtypeStruct((_B_SC,), jnp.float32),
    mesh=plsc.VectorSubcoreMesh(core_axis_name="c", subcore_axis_name="s"),
    compiler_params=pltpu.CompilerParams(needs_layout_passes=False),
    scratch_types=[
        pltpu.VMEM((_T, _D), jnp.float32),
        pltpu.VMEM((_T, _D), jnp.float32),
        pltpu.VMEM((_T,), jnp.float32),
        pltpu.VMEM_SHARED((_K, 2 * _D), jnp.float32),
        pltpu.VMEM_SHARED((1, 16), jnp.float32),
        pltpu.SMEM((_K, 2 * _D), jnp.float32),
        pltpu.SMEM((1, 16), jnp.float32),
        pltpu.SemaphoreType.DMA,
        pltpu.SemaphoreType.DMA,
    ],
)
def _sc_kernel(x_hbm, ab_hbm, c_hbm, out_hbm, *scratch):
    _sc_body(x_hbm, ab_hbm, c_hbm, out_hbm, *scratch)


def _tc_body(x_ref, locs_ref, logscales_ref, logcoefs_ref, out_ref):
    locs = locs_ref[...]
    logscales = logscales_ref[...]
    logcoefs = logcoefs_ref[...]

    scale = jnp.exp(logscales) + _MIN_SCALE
    inv2 = 1.0 / (scale * scale)
    a = -0.5 * inv2
    t = locs * inv2
    lc = logcoefs[0] - jax.nn.logsumexp(logcoefs[0])
    c = (lc
         - jnp.sum(jnp.log(scale), axis=1)
         - 0.5 * _D * _LOG2PI
         - 0.5 * jnp.sum(locs * locs * inv2, axis=1))

    x = x_ref[...]
    nt = (((1,), (1,)), ((), ()))
    lp = (lax.dot_general(t, x, nt, preferred_element_type=jnp.float32)
          + lax.dot_general(a, x * x, nt, preferred_element_type=jnp.float32)
          + c[:, None])
    m = jnp.max(lp, axis=0)
    s = jnp.sum(jnp.exp(lp - m[None, :]), axis=0)
    out_ref[...] = (m + jnp.log(s))[None, None, :]


def _tc_part(x, locs, logscales, logcoefs):
    grid = _B_TC // _BLK
    out2d = pl.pallas_call(
        _tc_body,
        grid=(grid,),
        in_specs=[
            pl.BlockSpec((_BLK, _D), lambda i: (i, 0)),
            pl.BlockSpec((_K, _D), lambda i: (0, 0)),
            pl.BlockSpec((_K, _D), lambda i: (0, 0)),
            pl.BlockSpec((1, _K), lambda i: (0, 0)),
        ],
        out_specs=pl.BlockSpec((1, 1, _BLK), lambda i: (i, 0, 0)),
        out_shape=jax.ShapeDtypeStruct((grid, 1, _BLK), jnp.float32),
    )(x, locs, logscales, logcoefs)
    return out2d.reshape(_B_TC)


@jax.jit
def kernel(x, locs, logscales, logcoefs):
    ab, c = _prep(locs, logscales, logcoefs)
    out_sc = _sc_kernel(x, ab, c)
    out_tc = _tc_part(x, locs, logscales, logcoefs)
    return jnp.concatenate([out_tc, out_sc])


# speedup vs baseline: 1.9937x; 1.5663x over previous
import functools
import math

import jax
import jax.numpy as jnp
from jax import lax
from jax.experimental import pallas as pl
from jax.experimental.pallas import tpu as pltpu
from jax.experimental.pallas import tpu_sc as plsc

_B = 262144
_D = 64
_K = 8
_MIN_SCALE = 1e-10
_LOG2PI = math.log(2.0 * math.pi)
_LN2 = math.log(2.0)

_NW = 32
_B_SC = 49152
_B_TC = _B - _B_SC
_ROWS_PER_W = _B_SC // _NW
_T = 256
_NT = _ROWS_PER_W // _T
_BLK = 16384

_LOG_POLY = (
    3.3423269089893903e-08, 1.000003098647089, -0.5000129330593959,
    0.33304812395033884, -0.24911210645380452, 0.2061178523941565,
    -0.18627697325890152, 0.11448435453731831,
)


def _prep_body(locs_ref, logscales_ref, logcoefs_ref, ab_ref, c_ref):
    locs = locs_ref[...]
    logscales = logscales_ref[...]
    logcoefs = logcoefs_ref[...]
    scale = jnp.exp(logscales) + _MIN_SCALE
    inv2 = 1.0 / (scale * scale)
    ab_ref[...] = jnp.concatenate([-0.5 * inv2, locs * inv2], axis=1)
    lc = logcoefs[0] - jax.nn.logsumexp(logcoefs[0])
    c = (lc
         - jnp.sum(jnp.log(scale), axis=1)
         - 0.5 * _D * _LOG2PI
         - 0.5 * jnp.sum(locs * locs * inv2, axis=1))
    c_ref[...] = jnp.concatenate([c, jnp.zeros((8,), jnp.float32)])[None, :]


def _prep(locs, logscales, logcoefs):
    return pl.pallas_call(
        _prep_body,
        out_shape=(
            jax.ShapeDtypeStruct((_K, 2 * _D), jnp.float32),
            jax.ShapeDtypeStruct((1, 16), jnp.float32),
        ),
    )(locs, logscales, logcoefs)


def _poly_log(s):
    bits = lax.bitcast_convert_type(s, jnp.int32)
    e = lax.shift_right_logical(bits, 23) - 127
    mant = lax.bitwise_or(lax.bitwise_and(bits, 0x007FFFFF), 0x3F800000)
    u = lax.bitcast_convert_type(mant, jnp.float32)
    big = u > 1.4142135
    u = jnp.where(big, u * 0.5, u)
    e = jnp.where(big, e + 1, e)
    w = u - 1.0
    p = jnp.full_like(w, _LOG_POLY[-1])
    for coef in _LOG_POLY[-2::-1]:
        p = p * w + coef
    return e.astype(jnp.float32) * _LN2 + p


def _sc_body(x_hbm, ab_hbm, c_hbm, out_hbm,
             xv0, xv1, outv, ab_sp, c_sp, ab_sm, c_sm, sem0, sem1):
    cid = lax.axis_index("c")
    sid = lax.axis_index("s")
    wid = sid * 2 + cid
    base = _B_TC + wid * _ROWS_PER_W

    @pl.when(sid == 0)
    def _():
        pltpu.sync_copy(ab_hbm, ab_sp)
        pltpu.sync_copy(c_hbm, c_sp)

    plsc.subcore_barrier()
    pltpu.sync_copy(ab_sp, ab_sm)
    pltpu.sync_copy(c_sp, c_sm)

    iota = lax.iota(jnp.int32, 16)
    bufs = (xv0, xv1)
    sems = (sem0, sem1)

    def start(t):
        return pltpu.async_copy(
            x_hbm.at[pl.ds(base + t * _T, _T), :],
            bufs[t % 2], sems[t % 2])

    pending = start(0)
    for t in range(_NT):
        pending.wait()
        if t + 1 < _NT:
            pending = start(t + 1)
        buf = bufs[t % 2]

        def group(g, carry):
            rows = g * 16 + iota

            def dstep(d, accs):
                cols = jnp.full((16,), d, jnp.int32)
                xv = plsc.load_gather(buf, [rows, cols])
                x2 = xv * xv
                return tuple(
                    accs[i] + ab_sm[i, d] * x2 + ab_sm[i, _D + d] * xv
                    for i in range(_K))

            init = tuple(jnp.full((16,), c_sm[0, i], jnp.float32)
                         for i in range(_K))
            accs = lax.fori_loop(0, _D, dstep, init, unroll=8)
            m01 = jnp.maximum(accs[0], accs[1])
            m23 = jnp.maximum(accs[2], accs[3])
            m45 = jnp.maximum(accs[4], accs[5])
            m67 = jnp.maximum(accs[6], accs[7])
            m = jnp.maximum(jnp.maximum(m01, m23), jnp.maximum(m45, m67))
            s = jnp.zeros((16,), jnp.float32)
            for i in range(_K):
                s = s + jnp.exp(accs[i] - m)
            outv[pl.ds(g * 16, 16)] = m + _poly_log(s)
            return carry

        lax.fori_loop(0, _T // 16, group, 0)
        pltpu.sync_copy(outv, out_hbm.at[pl.ds(wid * _ROWS_PER_W + t * _T, _T)])


@functools.partial(
    pl.kernel,
    out_type=jax.ShapeDtypeStruct((_B_SC,), jnp.float32),
    mesh=plsc.VectorSubcoreMesh(core_axis_name="c", subcore_axis_name="s"),
    compiler_params=pltpu.CompilerParams(needs_layout_passes=False),
    scratch_types=[
        pltpu.VMEM((_T, _D), jnp.float32),
        pltpu.VMEM((_T, _D), jnp.float32),
        pltpu.VMEM((_T,), jnp.float32),
        pltpu.VMEM_SHARED((_K, 2 * _D), jnp.float32),
        pltpu.VMEM_SHARED((1, 16), jnp.float32),
        pltpu.SMEM((_K, 2 * _D), jnp.float32),
        pltpu.SMEM((1, 16), jnp.float32),
        pltpu.SemaphoreType.DMA,
        pltpu.SemaphoreType.DMA,
    ],
)
def _sc_kernel(x_hbm, ab_hbm, c_hbm, out_hbm, *scratch):
    _sc_body(x_hbm, ab_hbm, c_hbm, out_hbm, *scratch)


def _tc_body(x_ref, locs_ref, logscales_ref, logcoefs_ref, out_ref):
    locs = locs_ref[...]
    logscales = logscales_ref[...]
    logcoefs = logcoefs_ref[...]

    scale = jnp.exp(logscales) + _MIN_SCALE
    inv2 = 1.0 / (scale * scale)
    a = -0.5 * inv2
    t = locs * inv2
    lc = logcoefs[0] - jax.nn.logsumexp(logcoefs[0])
    c = (lc
         - jnp.sum(jnp.log(scale), axis=1)
         - 0.5 * _D * _LOG2PI
         - 0.5 * jnp.sum(locs * locs * inv2, axis=1))

    x = x_ref[...]
    nt = (((1,), (1,)), ((), ()))
    lp = (lax.dot_general(t, x, nt, preferred_element_type=jnp.float32)
          + lax.dot_general(a, x * x, nt, preferred_element_type=jnp.float32)
          + c[:, None])
    m = jnp.max(lp, axis=0)
    s = jnp.sum(jnp.exp(lp - m[None, :]), axis=0)
    out_ref[...] = (m + jnp.log(s))[None, None, :]


def _tc_part(x, locs, logscales, logcoefs):
    grid = _B_TC // _BLK
    out2d = pl.pallas_call(
        _tc_body,
        grid=(grid,),
        in_specs=[
            pl.BlockSpec((_BLK, _D), lambda i: (i, 0)),
            pl.BlockSpec((_K, _D), lambda i: (0, 0)),
            pl.BlockSpec((_K, _D), lambda i: (0, 0)),
            pl.BlockSpec((1, _K), lambda i: (0, 0)),
        ],
        out_specs=pl.BlockSpec((1, 1, _BLK), lambda i: (i, 0, 0)),
        out_shape=jax.ShapeDtypeStruct((grid, 1, _BLK), jnp.float32),
    )(x, locs, logscales, logcoefs)
    return out2d.reshape(_B_TC)


def _tc_manual_body(x_hbm, locs_ref, logscales_ref, logcoefs_ref, out_ref,
                    xbuf, sems):
    i = pl.program_id(0)
    n = pl.num_programs(0)

    def start(step, slot):
        pltpu.make_async_copy(
            x_hbm.at[pl.ds(step * _BLK, _BLK), :],
            xbuf.at[slot], sems.at[slot]).start()

    @pl.when(i == 0)
    def _():
        start(0, 0)

    @pl.when(i + 1 < n)
    def _():
        start(i + 1, (i + 1) % 2)

    slot = i % 2
    pltpu.make_async_copy(
        x_hbm.at[pl.ds(i * _BLK, _BLK), :],
        xbuf.at[slot], sems.at[slot]).wait()

    locs = locs_ref[...]
    logscales = logscales_ref[...]
    logcoefs = logcoefs_ref[...]

    scale = jnp.exp(logscales) + _MIN_SCALE
    inv2 = 1.0 / (scale * scale)
    a = -0.5 * inv2
    t = locs * inv2
    lc = logcoefs[0] - jax.nn.logsumexp(logcoefs[0])
    c = (lc
         - jnp.sum(jnp.log(scale), axis=1)
         - 0.5 * _D * _LOG2PI
         - 0.5 * jnp.sum(locs * locs * inv2, axis=1))

    x = xbuf[slot]
    nt = (((1,), (1,)), ((), ()))
    lp = (lax.dot_general(t, x, nt, preferred_element_type=jnp.float32)
          + lax.dot_general(a, x * x, nt, preferred_element_type=jnp.float32)
          + c[:, None])
    m = jnp.max(lp, axis=0)
    s = jnp.sum(jnp.exp(lp - m[None, :]), axis=0)
    out_ref[...] = (m + jnp.log(s))[None, None, :]


def _tc_manual(x, locs, logscales, logcoefs, rows):
    grid = rows // _BLK
    out2d = pl.pallas_call(
        _tc_manual_body,
        grid=(grid,),
        in_specs=[
            pl.BlockSpec(memory_space=pltpu.MemorySpace.HBM),
            pl.BlockSpec((_K, _D), lambda i: (0, 0)),
            pl.BlockSpec((_K, _D), lambda i: (0, 0)),
            pl.BlockSpec((1, _K), lambda i: (0, 0)),
        ],
        out_specs=pl.BlockSpec((1, 1, _BLK), lambda i: (i, 0, 0)),
        out_shape=jax.ShapeDtypeStruct((grid, 1, _BLK), jnp.float32),
        scratch_shapes=[
            pltpu.VMEM((2, _BLK, _D), jnp.float32),
            pltpu.SemaphoreType.DMA((2,)),
        ],
    )(x, locs, logscales, logcoefs)
    return out2d.reshape(rows)


@jax.jit
def kernel(x, locs, logscales, logcoefs):
    return _tc_manual(x, locs, logscales, logcoefs, _B)
